# baseline (device time: 41548 ns/iter reference)
import jax
import jax.numpy as jnp
from jax import lax
from jax.experimental import pallas as pl
from jax.experimental.pallas import tpu as pltpu


def kernel(x, assign, W1, W2):
    t, d = x.shape
    e_per = W1.shape[0]
    assign2 = assign.reshape(t, 1)

    def body(x_ref, a_ref, w1_ref, w2_ref, out_ref,
             xr_ref, ar_ref, rr_ref, b_ref, send_sems, recv_sems):
        my_x = lax.axis_index("x")
        my_y = lax.axis_index("y")
        my_z = lax.axis_index("z")
        partner = (1 - my_x, my_y, my_z)

        barrier_sem = pltpu.get_barrier_semaphore()
        pl.semaphore_signal(barrier_sem, inc=1, device_id=partner,
                            device_id_type=pl.DeviceIdType.MESH)
        pl.semaphore_wait(barrier_sem, 1)

        rdma_x = pltpu.make_async_remote_copy(
            src_ref=x_ref, dst_ref=xr_ref,
            send_sem=send_sems.at[0], recv_sem=recv_sems.at[0],
            device_id=partner, device_id_type=pl.DeviceIdType.MESH)
        rdma_a = pltpu.make_async_remote_copy(
            src_ref=a_ref, dst_ref=ar_ref,
            send_sem=send_sems.at[1], recv_sem=recv_sems.at[1],
            device_id=partner, device_id_type=pl.DeviceIdType.MESH)
        rdma_x.start()
        rdma_a.start()

        def my_experts_contrib(xs, am):
            acc = jnp.zeros((t, d), jnp.float32)
            for e_loc in range(e_per):
                ge = e_loc + e_per * my_x
                h = jnp.maximum(
                    jnp.dot(xs, w1_ref[e_loc],
                            preferred_element_type=jnp.float32), 0.0)
                y = jnp.dot(h, w2_ref[e_loc],
                            preferred_element_type=jnp.float32)
                acc = acc + (am == ge).astype(jnp.float32) * y
            return acc

        out_ref[:, :] = my_experts_contrib(x_ref[:, :], a_ref[:, :])

        rdma_x.wait_recv()
        rdma_a.wait_recv()

        b_ref[:, :] = my_experts_contrib(xr_ref[:, :], ar_ref[:, :])
        rdma_b = pltpu.make_async_remote_copy(
            src_ref=b_ref, dst_ref=rr_ref,
            send_sem=send_sems.at[2], recv_sem=recv_sems.at[2],
            device_id=partner, device_id_type=pl.DeviceIdType.MESH)
        rdma_b.start()
        rdma_b.wait_recv()

        out_ref[:, :] = out_ref[:, :] + rr_ref[:, :]

        rdma_x.wait_send()
        rdma_a.wait_send()
        rdma_b.wait_send()

    return pl.pallas_call(
        body,
        out_shape=jax.ShapeDtypeStruct((t, d), jnp.float32),
        in_specs=[pl.BlockSpec(memory_space=pltpu.VMEM)] * 4,
        out_specs=pl.BlockSpec(memory_space=pltpu.VMEM),
        scratch_shapes=[
            pltpu.VMEM((t, d), jnp.float32),
            pltpu.VMEM((t, 1), jnp.int32),
            pltpu.VMEM((t, d), jnp.float32),
            pltpu.VMEM((t, d), jnp.float32),
            pltpu.SemaphoreType.DMA((3,)),
            pltpu.SemaphoreType.DMA((3,)),
        ],
        compiler_params=pltpu.CompilerParams(collective_id=0),
    )(x, assign2, W1, W2)


# device time: 29219 ns/iter; 1.4220x vs baseline; 1.4220x over previous
import jax
import jax.numpy as jnp
from jax import lax
from jax.experimental import pallas as pl
from jax.experimental.pallas import tpu as pltpu


def kernel(x, assign, W1, W2):
    t, d = x.shape
    e_per = W1.shape[0]
    assign2 = assign.reshape(t, 1)
    x16 = x.astype(jnp.bfloat16)
    W1_16 = W1.astype(jnp.bfloat16)
    W2_16 = W2.astype(jnp.bfloat16)

    def body(x_ref, a_ref, w1_ref, w2_ref, out_ref,
             xr_ref, ar_ref, rr_ref, b_ref, send_sems, recv_sems):
        my_x = lax.axis_index("x")
        my_y = lax.axis_index("y")
        my_z = lax.axis_index("z")
        partner = (1 - my_x, my_y, my_z)

        barrier_sem = pltpu.get_barrier_semaphore()
        pl.semaphore_signal(barrier_sem, inc=1, device_id=partner,
                            device_id_type=pl.DeviceIdType.MESH)
        pl.semaphore_wait(barrier_sem, 1)

        rdma_x = pltpu.make_async_remote_copy(
            src_ref=x_ref, dst_ref=xr_ref,
            send_sem=send_sems.at[0], recv_sem=recv_sems.at[0],
            device_id=partner, device_id_type=pl.DeviceIdType.MESH)
        rdma_a = pltpu.make_async_remote_copy(
            src_ref=a_ref, dst_ref=ar_ref,
            send_sem=send_sems.at[1], recv_sem=recv_sems.at[1],
            device_id=partner, device_id_type=pl.DeviceIdType.MESH)
        rdma_x.start()
        rdma_a.start()

        def my_experts_contrib(xs, am):
            acc = jnp.zeros((t, d), jnp.float32)
            for e_loc in range(e_per):
                ge = e_loc + e_per * my_x
                h = jnp.maximum(
                    jnp.dot(xs, w1_ref[e_loc],
                            preferred_element_type=jnp.float32), 0.0)
                y = jnp.dot(h.astype(jnp.bfloat16), w2_ref[e_loc],
                            preferred_element_type=jnp.float32)
                acc = acc + (am == ge).astype(jnp.float32) * y
            return acc

        out_ref[:, :] = my_experts_contrib(x_ref[:, :], a_ref[:, :])

        rdma_x.wait_recv()
        rdma_a.wait_recv()

        b_ref[:, :] = my_experts_contrib(
            xr_ref[:, :], ar_ref[:, :]).astype(jnp.bfloat16)
        rdma_b = pltpu.make_async_remote_copy(
            src_ref=b_ref, dst_ref=rr_ref,
            send_sem=send_sems.at[2], recv_sem=recv_sems.at[2],
            device_id=partner, device_id_type=pl.DeviceIdType.MESH)
        rdma_b.start()
        rdma_b.wait_recv()

        out_ref[:, :] = out_ref[:, :] + rr_ref[:, :].astype(jnp.float32)

        rdma_x.wait_send()
        rdma_a.wait_send()
        rdma_b.wait_send()

    return pl.pallas_call(
        body,
        out_shape=jax.ShapeDtypeStruct((t, d), jnp.float32),
        in_specs=[pl.BlockSpec(memory_space=pltpu.VMEM)] * 4,
        out_specs=pl.BlockSpec(memory_space=pltpu.VMEM),
        scratch_shapes=[
            pltpu.VMEM((t, d), jnp.bfloat16),
            pltpu.VMEM((t, 1), jnp.int32),
            pltpu.VMEM((t, d), jnp.bfloat16),
            pltpu.VMEM((t, d), jnp.bfloat16),
            pltpu.SemaphoreType.DMA((3,)),
            pltpu.SemaphoreType.DMA((3,)),
        ],
        compiler_params=pltpu.CompilerParams(collective_id=0),
    )(x16, assign2, W1_16, W2_16)
